# baseline (speedup 1.0000x reference)
import functools

import jax
import jax.numpy as jnp
from jax import lax
from jax.experimental import pallas as pl
from jax.experimental.pallas import tpu as pltpu
from jax.experimental.pallas import tpu_sc as plsc

_NC, _NS = 2, 16
_NW = _NC * _NS
_ROWS, _COLS, _OUTC = 16384, 2048, 128
_STRIDE = _COLS // _OUTC
_R = 16
_ROWS_W = _ROWS // _NW
_CHUNKS = _ROWS_W // _R
_NBUF = 2

_mesh = plsc.VectorSubcoreMesh(core_axis_name="c", subcore_axis_name="s")


@functools.partial(
    pl.kernel,
    out_type=jax.ShapeDtypeStruct((_ROWS, _OUTC), jnp.float32),
    mesh=_mesh,
    scratch_types=[
        [pltpu.VMEM((_R, _COLS), jnp.float32) for _ in range(_NBUF)],
        [pltpu.VMEM((_R, _OUTC), jnp.float32) for _ in range(_NBUF)],
        [pltpu.SemaphoreType.DMA for _ in range(_NBUF)],
        [pltpu.SemaphoreType.DMA for _ in range(_NBUF)],
    ],
    compiler_params=pltpu.CompilerParams(needs_layout_passes=False),
)
def _select(x_hbm, out_hbm, xins, youts, sis, sos):
    wid = lax.axis_index("s") * _NC + lax.axis_index("c")
    row0 = wid * _ROWS_W
    lane = lax.iota(jnp.int32, 16)
    col_sel = lane * _STRIDE

    def in_slice(g):
        return x_hbm.at[pl.ds(row0 + g * _R, _R), :]

    def out_slice(g):
        return out_hbm.at[pl.ds(row0 + g * _R, _R), :]

    for b in range(_NBUF - 1):
        pltpu.async_copy(in_slice(b), xins[b], sis[b])

    def body(h, carry):
        for b in range(_NBUF):
            g = h * _NBUF + b
            pltpu.make_async_copy(in_slice(g), xins[b], sis[b]).wait()

            @pl.when(g + _NBUF - 1 < _CHUNKS)
            def _():
                nb = (b + _NBUF - 1) % _NBUF
                pltpu.async_copy(in_slice(g + _NBUF - 1), xins[nb], sis[nb])

            @pl.when(g >= _NBUF)
            def _():
                pltpu.make_async_copy(youts[b], out_slice(g - _NBUF), sos[b]).wait()

            for r in range(_R):
                rvec = jnp.full((16,), r, jnp.int32)
                for v in range(_OUTC // 16):
                    idx_col = col_sel + (v * 16 * _STRIDE)
                    youts[b][r, pl.ds(v * 16, 16)] = plsc.load_gather(
                        xins[b], [rvec, idx_col]
                    )

            pltpu.async_copy(youts[b], out_slice(g), sos[b])
        return carry

    lax.fori_loop(0, _CHUNKS // _NBUF, body, 0)

    for b in range(_NBUF):
        g = _CHUNKS - _NBUF + b
        pltpu.make_async_copy(youts[b], out_slice(g), sos[b]).wait()


def kernel(x):
    return _select(x)


# speedup vs baseline: 1.3625x; 1.3625x over previous
import functools

import jax
import jax.numpy as jnp
from jax import lax
from jax.experimental import pallas as pl
from jax.experimental.pallas import tpu as pltpu
from jax.experimental.pallas import tpu_sc as plsc

_NC, _NS = 2, 16
_NW = _NC * _NS
_ROWS, _COLS, _OUTC = 16384, 2048, 128
_STRIDE = _COLS // _OUTC
_R = 4
_ROWS_W = _ROWS // _NW
_CHUNKS = _ROWS_W // _R
_NBUF = 8

_mesh = plsc.VectorSubcoreMesh(core_axis_name="c", subcore_axis_name="s")


@functools.partial(
    pl.kernel,
    out_type=jax.ShapeDtypeStruct((_ROWS, _OUTC), jnp.float32),
    mesh=_mesh,
    scratch_types=[
        [pltpu.VMEM((_R, _COLS), jnp.float32) for _ in range(_NBUF)],
        [pltpu.VMEM((_R, _OUTC), jnp.float32) for _ in range(_NBUF)],
        [pltpu.SemaphoreType.DMA for _ in range(_NBUF)],
        [pltpu.SemaphoreType.DMA for _ in range(_NBUF)],
    ],
    compiler_params=pltpu.CompilerParams(needs_layout_passes=False),
)
def _select(x_hbm, out_hbm, xins, youts, sis, sos):
    wid = lax.axis_index("s") * _NC + lax.axis_index("c")
    row0 = wid * _ROWS_W
    lane = lax.iota(jnp.int32, 16)
    col_sel = lane * _STRIDE

    def in_slice(g):
        return x_hbm.at[pl.ds(row0 + g * _R, _R), :]

    def out_slice(g):
        return out_hbm.at[pl.ds(row0 + g * _R, _R), :]

    for b in range(_NBUF - 1):
        pltpu.async_copy(in_slice(b), xins[b], sis[b])

    def body(h, carry):
        for b in range(_NBUF):
            g = h * _NBUF + b
            pltpu.make_async_copy(in_slice(g), xins[b], sis[b]).wait()

            @pl.when(g + _NBUF - 1 < _CHUNKS)
            def _():
                nb = (b + _NBUF - 1) % _NBUF
                pltpu.async_copy(in_slice(g + _NBUF - 1), xins[nb], sis[nb])

            @pl.when(g >= _NBUF)
            def _():
                pltpu.make_async_copy(youts[b], out_slice(g - _NBUF), sos[b]).wait()

            for r in range(_R):
                rvec = jnp.full((16,), r, jnp.int32)
                for v in range(_OUTC // 16):
                    idx_col = col_sel + (v * 16 * _STRIDE)
                    youts[b][r, pl.ds(v * 16, 16)] = plsc.load_gather(
                        xins[b], [rvec, idx_col]
                    )

            pltpu.async_copy(youts[b], out_slice(g), sos[b])
        return carry

    lax.fori_loop(0, _CHUNKS // _NBUF, body, 0)

    for b in range(_NBUF):
        g = _CHUNKS - _NBUF + b
        pltpu.make_async_copy(youts[b], out_slice(g), sos[b]).wait()


def kernel(x):
    return _select(x)


# speedup vs baseline: 1.7113x; 1.2560x over previous
import functools

import jax
import jax.numpy as jnp
from jax import lax
from jax.experimental import pallas as pl
from jax.experimental.pallas import tpu as pltpu
from jax.experimental.pallas import tpu_sc as plsc

_NC, _NS = 2, 16
_NW = _NC * _NS
_ROWS, _COLS, _OUTC = 16384, 2048, 128
_STRIDE = _COLS // _OUTC
_R = 4
_ROWS_W = _ROWS // _NW
_CHUNKS = _ROWS_W // _R
_NBUF = 8

_mesh = plsc.VectorSubcoreMesh(core_axis_name="c", subcore_axis_name="s")


@functools.partial(
    pl.kernel,
    out_type=jax.ShapeDtypeStruct((_ROWS, _OUTC), jnp.float32),
    mesh=_mesh,
    scratch_types=[
        [pltpu.VMEM((_R, _COLS), jnp.float32) for _ in range(_NBUF)],
        [pltpu.VMEM((_R, _OUTC), jnp.float32) for _ in range(_NBUF)],
        [pltpu.SemaphoreType.DMA for _ in range(_NBUF)],
        [pltpu.SemaphoreType.DMA for _ in range(_NBUF)],
    ],
    compiler_params=pltpu.CompilerParams(needs_layout_passes=False),
)
def _select(x_hbm, out_hbm, xins, youts, sis, sos):
    wid = lax.axis_index("s") * _NC + lax.axis_index("c")
    row0 = wid * _ROWS_W
    lane = lax.iota(jnp.int32, 16)
    col_sel = lane * _STRIDE

    def in_slice(g):
        return x_hbm.at[pl.ds(row0 + g * _R, _R), :]

    def out_slice(g):
        return out_hbm.at[pl.ds(row0 + g * _R, _R), :]

    for b in range(_NBUF - 1):
        pltpu.async_copy(in_slice(b), xins[b], sis[b])

    def body(h, carry):
        for b in range(_NBUF):
            g = h * _NBUF + b
            pltpu.make_async_copy(in_slice(g), xins[b], sis[b]).wait()

            @pl.when(g + _NBUF - 1 < _CHUNKS)
            def _():
                nb = (b + _NBUF - 1) % _NBUF
                pltpu.async_copy(in_slice(g + _NBUF - 1), xins[nb], sis[nb])

            @pl.when(g >= _NBUF)
            def _():
                pltpu.make_async_copy(youts[b], out_slice(g - _NBUF), sos[b]).wait()

            for r in range(_R):
                rvec = jnp.full((16,), r, jnp.int32)
                for v in range(_OUTC // 16):
                    idx_col = col_sel + (v * 16 * _STRIDE)
                    youts[b][r, pl.ds(v * 16, 16)] = plsc.load_gather(
                        xins[b], [rvec, idx_col]
                    )

            pltpu.async_copy(youts[b], out_slice(g), sos[b])
        return carry

    lax.fori_loop(0, _CHUNKS // _NBUF, body, 0)

    for b in range(_NBUF):
        g = _CHUNKS - _NBUF + b
        pltpu.make_async_copy(youts[b], out_slice(g), sos[b]).wait()


_TC_BR = 512


def _tc_body(x_ref, s_ref, o_ref):
    o_ref[...] = jax.lax.dot(
        x_ref[...], s_ref[...], preferred_element_type=jnp.float32
    )


def _select_tc(x, sel):
    n = x.shape[0]
    return pl.pallas_call(
        _tc_body,
        grid=(n // _TC_BR,),
        in_specs=[
            pl.BlockSpec((_TC_BR, _COLS), lambda i: (i, 0)),
            pl.BlockSpec((_COLS, _OUTC), lambda i: (0, 0)),
        ],
        out_specs=pl.BlockSpec((_TC_BR, _OUTC), lambda i: (i, 0)),
        out_shape=jax.ShapeDtypeStruct((n, _OUTC), jnp.float32),
        compiler_params=pltpu.CompilerParams(
            dimension_semantics=("arbitrary",)
        ),
    )(x, sel)


def _selection_matrix():
    c = lax.broadcasted_iota(jnp.int32, (_COLS, _OUTC), 0)
    j = lax.broadcasted_iota(jnp.int32, (_COLS, _OUTC), 1)
    return (c == j * _STRIDE).astype(jnp.float32)


def kernel(x):
    return _select_tc(x, _selection_matrix())
